# baseline (speedup 1.0000x reference)
import functools

import jax
import jax.numpy as jnp
from jax import lax
from jax.experimental import pallas as pl
from jax.experimental.pallas import tpu as pltpu
from jax.experimental.pallas import tpu_sc as plsc

_IDX_CHUNK = 128


def _emb_gather(ids2d, padded_table, batch, dim):
    total_chunks, idx_chunk = ids2d.shape
    _, dim2 = padded_table.shape
    info = plsc.get_sparse_core_info()
    nw = info.num_cores * info.num_subcores
    n_chunks = total_chunks // nw
    rows_per_w = n_chunks * idx_chunk

    mesh = plsc.VectorSubcoreMesh(core_axis_name="c", subcore_axis_name="s")

    @functools.partial(
        pl.kernel,
        mesh=mesh,
        out_type=jax.ShapeDtypeStruct((batch, dim), jnp.float32),
        scratch_types=[
            pltpu.VMEM((n_chunks, idx_chunk), jnp.int32),
            pltpu.VMEM((rows_per_w, dim2), jnp.float32),
            pltpu.VMEM((2, idx_chunk, dim), jnp.float32),
            pltpu.SemaphoreType.DMA,
            pltpu.SemaphoreType.DMA,
        ],
    )
    def emb(idx_hbm, table_hbm, out_hbm, idx_v, rows_v, out_v, gsem, wsem):
        wid = lax.axis_index("s") * info.num_cores + lax.axis_index("c")
        pltpu.sync_copy(idx_hbm.at[pl.ds(wid * n_chunks, n_chunks)], idx_v)
        gathers = [
            pltpu.async_copy(
                table_hbm.at[idx_v.at[k]],
                rows_v.at[pl.ds(k * idx_chunk, idx_chunk)],
                gsem,
            )
            for k in range(n_chunks)
        ]
        writes = []
        for k in range(n_chunks):
            gathers[k].wait()
            if k >= 2:
                writes[k - 2].wait()

            for r in range(idx_chunk):
                for j in range(dim // 16):
                    out_v[k % 2, r, pl.ds(j * 16, 16)] = rows_v[
                        k * idx_chunk + r, pl.ds(j * 16, 16)
                    ]
            writes.append(
                pltpu.async_copy(
                    out_v.at[k % 2],
                    out_hbm.at[pl.ds(wid * rows_per_w + k * idx_chunk, idx_chunk)],
                    wsem,
                )
            )
        for w in writes[-2:]:
            w.wait()

    return emb(ids2d, padded_table)


def kernel(emotion_ids, embedding_weight):
    ids = emotion_ids.astype(jnp.int32)
    table = embedding_weight.astype(jnp.float32)
    vocab, dim = table.shape
    batch, = ids.shape

    padded_table = jnp.pad(table, ((0, 0), (0, 128 - dim)))
    ids2d = ids.reshape(batch // _IDX_CHUNK, _IDX_CHUNK)

    return _emb_gather(ids2d, padded_table, batch, dim)


# speedup vs baseline: 1.0415x; 1.0415x over previous
import functools

import jax
import jax.numpy as jnp
from jax import lax
from jax.experimental import pallas as pl
from jax.experimental.pallas import tpu as pltpu
from jax.experimental.pallas import tpu_sc as plsc

_IDX_CHUNK = 128


def _emb_gather(ids2d, table, batch, dim):
    total_chunks, idx_chunk = ids2d.shape
    info = plsc.get_sparse_core_info()
    nw = info.num_cores * info.num_subcores
    n_chunks = total_chunks // nw
    rows_per_w = n_chunks * idx_chunk

    mesh = plsc.VectorSubcoreMesh(core_axis_name="c", subcore_axis_name="s")

    @functools.partial(
        pl.kernel,
        mesh=mesh,
        out_type=jax.ShapeDtypeStruct((batch, dim), jnp.float32),
        scratch_types=[
            pltpu.VMEM((n_chunks, idx_chunk), jnp.int32),
            pltpu.VMEM((rows_per_w, dim), jnp.float32),
            pltpu.SemaphoreType.DMA,
            pltpu.SemaphoreType.DMA,
        ],
        compiler_params=pltpu.CompilerParams(use_tc_tiling_on_sc=False),
    )
    def emb(idx_hbm, table_hbm, out_hbm, idx_v, rows_v, gsem, wsem):
        wid = lax.axis_index("s") * info.num_cores + lax.axis_index("c")
        pltpu.sync_copy(idx_hbm.at[pl.ds(wid * n_chunks, n_chunks)], idx_v)
        gathers = [
            pltpu.async_copy(
                table_hbm.at[idx_v.at[k]],
                rows_v.at[pl.ds(k * idx_chunk, idx_chunk)],
                gsem,
            )
            for k in range(n_chunks)
        ]
        writes = []
        for k in range(n_chunks):
            gathers[k].wait()
            writes.append(
                pltpu.async_copy(
                    rows_v.at[pl.ds(k * idx_chunk, idx_chunk)],
                    out_hbm.at[pl.ds(wid * rows_per_w + k * idx_chunk, idx_chunk)],
                    wsem,
                )
            )
        for w in writes:
            w.wait()

    return emb(ids2d, table)


def kernel(emotion_ids, embedding_weight):
    ids = emotion_ids.astype(jnp.int32)
    table = embedding_weight.astype(jnp.float32)
    vocab, dim = table.shape
    batch, = ids.shape

    ids2d = ids.reshape(batch // _IDX_CHUNK, _IDX_CHUNK)
    return _emb_gather(ids2d, table, batch, dim)


# speedup vs baseline: 2.4615x; 2.3635x over previous
import functools

import jax
import jax.numpy as jnp
from jax import lax
from jax.experimental import pallas as pl
from jax.experimental.pallas import tpu as pltpu
from jax.experimental.pallas import tpu_sc as plsc

_IDX_CHUNK = 128


def _pair_gather(pair_ids_2d, pair_table):
    total_chunks, idx_chunk = pair_ids_2d.shape
    _, dim2 = pair_table.shape
    info = plsc.get_sparse_core_info()
    nw = info.num_cores * info.num_subcores
    n_chunks = total_chunks // nw
    rows_per_w = n_chunks * idx_chunk

    mesh = plsc.VectorSubcoreMesh(core_axis_name="c", subcore_axis_name="s")

    @functools.partial(
        pl.kernel,
        mesh=mesh,
        out_type=jax.ShapeDtypeStruct((total_chunks * idx_chunk, dim2), jnp.float32),
        scratch_types=[
            pltpu.VMEM((n_chunks, idx_chunk), jnp.int32),
            pltpu.VMEM((rows_per_w, dim2), jnp.float32),
            pltpu.SemaphoreType.DMA,
            pltpu.SemaphoreType.DMA,
        ],
    )
    def emb(idx_hbm, table_hbm, out_hbm, idx_v, rows_v, gsem, wsem):
        wid = lax.axis_index("s") * info.num_cores + lax.axis_index("c")
        pltpu.sync_copy(idx_hbm.at[pl.ds(wid * n_chunks, n_chunks)], idx_v)
        gathers = [
            pltpu.async_copy(
                table_hbm.at[idx_v.at[k]],
                rows_v.at[pl.ds(k * idx_chunk, idx_chunk)],
                gsem,
            )
            for k in range(n_chunks)
        ]
        writes = []
        for k in range(n_chunks):
            gathers[k].wait()
            writes.append(
                pltpu.async_copy(
                    rows_v.at[pl.ds(k * idx_chunk, idx_chunk)],
                    out_hbm.at[pl.ds(wid * rows_per_w + k * idx_chunk, idx_chunk)],
                    wsem,
                )
            )
        for w in writes:
            w.wait()

    return emb(pair_ids_2d, pair_table)


def kernel(emotion_ids, embedding_weight):
    ids = emotion_ids.astype(jnp.int32)
    table = embedding_weight.astype(jnp.float32)
    vocab, dim = table.shape
    batch, = ids.shape

    left = jnp.repeat(table, vocab, axis=0)
    right = jnp.tile(table, (vocab, 1))
    pair_table = jnp.concatenate([left, right], axis=1)

    x = ids.reshape(batch // 256, 256)
    pair_ids_2d = x[:, 0::2] * vocab + x[:, 1::2]

    out2 = _pair_gather(pair_ids_2d, pair_table)
    return out2.reshape(batch, dim)
